# baseline (speedup 1.0000x reference)
import functools

import jax
import jax.numpy as jnp
from jax.experimental import pallas as pl
from jax.experimental.pallas import tpu as pltpu


def _body(x_ref, y_ref, w_ref, t_ref, out_ref,
          emb_ref, a2_ref, acc_ref, mse_ref, *, nk, tk, k_total):
    k = pl.program_id(0)

    @pl.when(k == 0)
    def _init():
        x = x_ref[...]
        emb = jax.lax.dot_general(
            x, w_ref[...], (((1,), (0,)), ((), ())),
            preferred_element_type=jnp.float32,
            precision=jax.lax.Precision.HIGHEST)
        a2_ref[...] = jnp.sum(emb * emb, axis=1, keepdims=True)
        emb_ref[...] = (-2.0 * emb).astype(jnp.bfloat16)
        diff = x - y_ref[...]
        mse_ref[0, 0] = jnp.mean(diff * diff)
        acc_ref[...] = jnp.full_like(acc_ref, jnp.inf)

    tt = t_ref[...]
    b2 = jnp.sum(tt * tt, axis=1)[None, :]
    d = jax.lax.dot_general(
        emb_ref[...], tt.astype(jnp.bfloat16), (((1,), (1,)), ((), ())),
        preferred_element_type=jnp.float32)
    d2 = d + b2

    @pl.when(k < nk - 1)
    def _accum():
        acc_ref[...] = jnp.minimum(acc_ref[...], d2)

    @pl.when(k == nk - 1)
    def _fin():
        rem = k_total - (nk - 1) * tk
        col = jax.lax.broadcasted_iota(jnp.int32, (1, tk), 1)
        d2m = jnp.where(col < rem, d2, jnp.inf)
        acc = jnp.minimum(acc_ref[...], d2m)
        m = jnp.min(acc, axis=1, keepdims=True)
        md = jnp.sqrt(jnp.maximum(a2_ref[...] + m, 0.0))
        out_ref[...] = mse_ref[0, 0] + jnp.minimum(50.0 - md, 0.0) * 5.0


@jax.jit
def kernel(x, y, W, table):
    q, d_in = x.shape
    k_total, d_emb = table.shape
    tk = 2048
    nk = pl.cdiv(k_total, tk)

    out = pl.pallas_call(
        functools.partial(_body, nk=nk, tk=tk, k_total=k_total),
        grid=(nk,),
        in_specs=[
            pl.BlockSpec((q, d_in), lambda k: (0, 0)),
            pl.BlockSpec((q, d_in), lambda k: (0, 0)),
            pl.BlockSpec((d_in, d_emb), lambda k: (0, 0)),
            pl.BlockSpec((tk, d_emb), lambda k: (k, 0)),
        ],
        out_specs=pl.BlockSpec((q, 1), lambda k: (0, 0)),
        out_shape=jax.ShapeDtypeStruct((q, 1), jnp.float32),
        scratch_shapes=[
            pltpu.VMEM((q, d_emb), jnp.bfloat16),
            pltpu.VMEM((q, 1), jnp.float32),
            pltpu.VMEM((q, tk), jnp.float32),
            pltpu.SMEM((1, 1), jnp.float32),
        ],
        compiler_params=pltpu.CompilerParams(
            dimension_semantics=("arbitrary",)),
    )(x, y, W, table)
    return out.reshape(q)


# speedup vs baseline: 1.5696x; 1.5696x over previous
import functools

import jax
import jax.numpy as jnp
from jax.experimental import pallas as pl
from jax.experimental.pallas import tpu as pltpu


def _body(x_ref, y_ref, w_ref, t_ref, out_ref,
          emb_ref, a2_ref, acc_ref, mse_ref, *, nk, tk, k_total):
    k = pl.program_id(0)

    @pl.when(k == 0)
    def _init():
        x = x_ref[...]
        emb = jax.lax.dot_general(
            x, w_ref[...], (((1,), (0,)), ((), ())),
            preferred_element_type=jnp.float32,
            precision=jax.lax.Precision.HIGHEST)
        a2_ref[...] = jnp.sum(emb * emb, axis=1, keepdims=True)
        emb_ref[...] = (-2.0 * emb).astype(jnp.bfloat16)
        diff = x - y_ref[...]
        mse_ref[0, 0] = jnp.mean(diff * diff)
        acc_ref[...] = jnp.full_like(acc_ref, jnp.inf)

    tt = t_ref[...]
    b2 = jnp.sum(tt * tt, axis=1)[None, :]
    d = jax.lax.dot_general(
        emb_ref[...], tt.astype(jnp.bfloat16), (((1,), (1,)), ((), ())),
        preferred_element_type=jnp.float32)
    d2 = d + b2
    acc_ref[...] = jnp.minimum(acc_ref[...],
                               jnp.min(d2, axis=1, keepdims=True))

    @pl.when(k == nk - 1)
    def _fin():
        md = jnp.sqrt(jnp.maximum(a2_ref[...] + acc_ref[...], 0.0))
        out_ref[...] = mse_ref[0, 0] + jnp.minimum(50.0 - md, 0.0) * 5.0


@jax.jit
def kernel(x, y, W, table):
    q, d_in = x.shape
    k_total, d_emb = table.shape
    tk = 4000
    nk = k_total // tk
    assert nk * tk == k_total

    out = pl.pallas_call(
        functools.partial(_body, nk=nk, tk=tk, k_total=k_total),
        grid=(nk,),
        in_specs=[
            pl.BlockSpec((q, d_in), lambda k: (0, 0)),
            pl.BlockSpec((q, d_in), lambda k: (0, 0)),
            pl.BlockSpec((d_in, d_emb), lambda k: (0, 0)),
            pl.BlockSpec((tk, d_emb), lambda k: (k, 0)),
        ],
        out_specs=pl.BlockSpec((q, 1), lambda k: (0, 0)),
        out_shape=jax.ShapeDtypeStruct((q, 1), jnp.float32),
        scratch_shapes=[
            pltpu.VMEM((q, d_emb), jnp.bfloat16),
            pltpu.VMEM((q, 1), jnp.float32),
            pltpu.VMEM((q, 1), jnp.float32),
            pltpu.SMEM((1, 1), jnp.float32),
        ],
        compiler_params=pltpu.CompilerParams(
            dimension_semantics=("arbitrary",)),
    )(x, y, W, table)
    return out.reshape(q)
